# baseline (speedup 1.0000x reference)
import jax
import jax.numpy as jnp
from jax.experimental import pallas as pl

N = 4096
C = 256
K = 16
R = 128
NBLK = N // R
NEG = float("-inf")
POS = float("inf")


def _body(xb_blk, xbt, aa_blk, aan, ncols, feats_blk, out_ref):
    f = feats_blk[0]
    aa_i = aa_blk[0]
    aa_n = aan[0]

    ab = jnp.dot(xb_blk[0], xbt[0], preferred_element_type=jnp.float32)
    dist = (aa_i + aa_n) - 2.0 * ab

    nc = ncols[0]

    cur = dist
    m = jnp.full((R, 1), NEG, dtype=jnp.float32)
    for r in range(K + 1):
        mn = jnp.min(cur, axis=1, keepdims=True)
        mask = cur == mn
        if r >= 1:
            col = nc[r - 1 : r, :]
            val = jnp.max(jnp.where(mask, col, NEG), axis=1, keepdims=True)
            m = jnp.maximum(m, val)
        cur = jnp.where(mask, POS, cur)
    n_erased = jnp.sum(jnp.where(cur == POS, 1.0, 0.0))
    any_dup = n_erased > (K + 1) * R + 0.5

    def exact_path(_):
        iota = jax.lax.broadcasted_iota(jnp.int32, (R, N), 1)
        ab2 = jnp.dot(xb_blk[0], xbt[0], preferred_element_type=jnp.float32)
        cur = (aa_i + aa_n) - 2.0 * ab2
        m = jnp.full((R, 1), NEG, dtype=jnp.float32)
        for r in range(K + 1):
            mn = jnp.min(cur, axis=1, keepdims=True)
            sel = jnp.min(jnp.where(cur == mn, iota, N), axis=1, keepdims=True)
            onehot = iota == sel
            if r >= 1:
                col = nc[r - 1 : r, :]
                val = jnp.max(jnp.where(onehot, col, NEG), axis=1, keepdims=True)
                m = jnp.maximum(m, val)
            cur = jnp.where(onehot, POS, cur)
        return m

    m = jax.lax.cond(any_dup, exact_path, lambda _: m, None)

    out_ref[0, :, :C] = f
    out_ref[0, :, C:] = m - f


@jax.jit
def _run(feats2, xb2, xbt2, aa2, aan2, ncols2):
    return pl.pallas_call(
        _body,
        grid=(2, NBLK),
        in_specs=[
            pl.BlockSpec((1, R, 3), lambda t, b: (t, b, 0)),
            pl.BlockSpec((1, 3, N), lambda t, b: (t, 0, 0)),
            pl.BlockSpec((1, R, 1), lambda t, b: (t, b, 0)),
            pl.BlockSpec((1, 1, N), lambda t, b: (t, 0, 0)),
            pl.BlockSpec((1, K, N), lambda t, b: (t, 0, 0)),
            pl.BlockSpec((1, R, C), lambda t, b: (t, b, 0)),
        ],
        out_specs=pl.BlockSpec((1, R, 2 * C), lambda t, b: (t, b, 0)),
        out_shape=jax.ShapeDtypeStruct((2, N, 2 * C), jnp.float32),
    )(xb2, xbt2, aa2, aan2, ncols2, feats2)


def kernel(src, tgt, src_coords, tgt_coords):
    feats2 = jnp.stack([src, tgt])
    coords2 = jnp.stack([src_coords, tgt_coords])
    aa2 = jnp.sum(coords2 * coords2, axis=2, keepdims=True)
    aan2 = jnp.transpose(aa2, (0, 2, 1))
    xb2 = coords2.astype(jnp.bfloat16)
    xbt2 = jnp.transpose(xb2, (0, 2, 1))
    ncols2 = jnp.transpose(feats2[:, :, :K], (0, 2, 1))
    out = _run(feats2, xb2, xbt2, aa2, aan2, ncols2)
    return out[0], out[1]
